# Initial kernel scaffold; baseline (speedup 1.0000x reference)
import jax
import jax.numpy as jnp
from jax.experimental import pallas as pl


def kernel(input_pos, cache_positions, seq_len):
    raise NotImplementedError("write your pallas kernel here")


# speedup vs baseline: 5.4497x; 5.4497x over previous
import jax
import jax.numpy as jnp
from jax import lax
from jax.experimental import pallas as pl
from jax.experimental.pallas import tpu as pltpu
from jax.experimental.pallas import tpu_sc as plsc

_MAX_CTX = 32768
_SEQ = 2048
_NC = 2
_NS = 16
_NW = _NC * _NS
_CHUNK = _MAX_CTX // _NW
_ICHUNK = _SEQ // _NW
_L = 16


def _body(params_hbm, cache_hbm, idx_hbm, out_hbm, pbuf, cbuf, obuf, ibuf):
    wid = lax.axis_index("s") * _NC + lax.axis_index("c")
    base = wid * _CHUNK
    ibase = wid * _ICHUNK

    pltpu.sync_copy(params_hbm, pbuf)
    pltpu.sync_copy(cache_hbm.at[pl.ds(base, _CHUNK)], cbuf)

    sp_vec = pbuf[pl.ds(0, _L)]
    st_vec = pbuf[pl.ds(_L, _L)]
    lane = lax.broadcasted_iota(jnp.int32, (_L,), 0)
    neg1 = jnp.full((_L,), -1, jnp.int32)

    for k in range(_CHUNK // _L):
        i_vec = lane + (base + k * _L)
        d = (i_vec - st_vec) & (_MAX_CTX - 1)
        cache_v = cbuf[pl.ds(k * _L, _L)]
        out = jnp.where(d < _SEQ, d + st_vec,
                        jnp.where(i_vec < sp_vec, cache_v, neg1))
        obuf[pl.ds(k * _L, _L)] = out

    for k in range(_ICHUNK // _L):
        j_vec = lane + (ibase + k * _L)
        ibuf[pl.ds(k * _L, _L)] = (j_vec + st_vec) & (_MAX_CTX - 1)

    pltpu.sync_copy(obuf, out_hbm.at[pl.ds(base, _CHUNK)])
    pltpu.sync_copy(ibuf, idx_hbm.at[pl.ds(ibase, _ICHUNK)])


def kernel(input_pos, cache_positions, seq_len):
    out_dtype = cache_positions.dtype
    start = input_pos[0].astype(jnp.int32)
    st = start + (jnp.asarray(seq_len).astype(jnp.int32) - _SEQ)
    params = jnp.concatenate(
        [jnp.broadcast_to(start, (_L,)), jnp.broadcast_to(st, (_L,))])
    cache32 = cache_positions.astype(jnp.int32)

    sc_call = pl.kernel(
        _body,
        out_type=(jax.ShapeDtypeStruct((_SEQ,), jnp.int32),
                  jax.ShapeDtypeStruct((_MAX_CTX,), jnp.int32)),
        mesh=plsc.VectorSubcoreMesh(core_axis_name="c", subcore_axis_name="s",
                                    num_cores=_NC, num_subcores=_NS),
        scratch_types=[
            pltpu.VMEM((2 * _L,), jnp.int32),
            pltpu.VMEM((_CHUNK,), jnp.int32),
            pltpu.VMEM((_CHUNK,), jnp.int32),
            pltpu.VMEM((_ICHUNK,), jnp.int32),
        ],
    )
    idx32, out32 = sc_call(params, cache32)
    return idx32.astype(out_dtype), out32.astype(out_dtype)
